# baseline (speedup 1.0000x reference)
import functools

import jax
import jax.numpy as jnp
from jax import lax
from jax.experimental import pallas as pl
from jax.experimental.pallas import tpu as pltpu
from jax.experimental.pallas import tpu_sc as plsc

N = 10000
E = 160000
HID = 256
HALF = 128
BN_EPS = 1e-5

BLK = 1000
NBLK = N // BLK

CHUNK = 64
NSUB = 16
NCHUNK_SUB = 160
NCHUNK_TOTAL = NSUB * NCHUNK_SUB
E_PAD = NCHUNK_TOTAL * CHUNK
EDGES_SUB = NCHUNK_SUB * CHUNK
ROWS_SUB = 632
AGG_ROWS = NSUB * ROWS_SUB
LAST_ROWS = N - 15 * ROWS_SUB


def _proj_body(x_ref, wp_ref, bp_ref, ha_ref, hb_ref):
    h = jnp.dot(x_ref[...].astype(jnp.bfloat16), wp_ref[...],
                preferred_element_type=jnp.float32)
    h = jnp.maximum(h + bp_ref[...], 0.0)
    ha_ref[...] = h[:, :HALF]
    hb_ref[...] = h[:, HALF:]


def _proj(x, Wp, bp2):
    return pl.pallas_call(
        _proj_body,
        grid=(NBLK,),
        in_specs=[
            pl.BlockSpec((BLK, HID), lambda i: (i, 0)),
            pl.BlockSpec((HID, HID), lambda i: (0, 0)),
            pl.BlockSpec((1, HID), lambda i: (0, 0)),
        ],
        out_specs=[
            pl.BlockSpec((BLK, HALF), lambda i: (i, 0)),
            pl.BlockSpec((BLK, HALF), lambda i: (i, 0)),
        ],
        out_shape=[
            jax.ShapeDtypeStruct((N, HALF), jnp.float32),
            jax.ShapeDtypeStruct((N, HALF), jnp.float32),
        ],
    )(x, Wp, bp2)


def _segsum_body(ha, hb, srcr, dstr, zr, agga, aggb,
                 srcf, dpage, *rest):
    rows = rest[0:4]
    aggsh = rest[4]
    sg = rest[5:9]
    ss = rest[9:13]
    c = lax.axis_index("c")
    s = lax.axis_index("s")

    pltpu.sync_copy(zr, rows[0])
    @pl.loop(0, ROWS_SUB // CHUNK)
    def _(i):
        pltpu.sync_copy(rows[0],
                        aggsh.at[pl.ds(s * ROWS_SUB + i * CHUNK, CHUNK)])
    tail = ROWS_SUB - (ROWS_SUB // CHUNK) * CHUNK
    if tail:
        pltpu.sync_copy(
            rows[0].at[pl.ds(0, tail)],
            aggsh.at[pl.ds(s * ROWS_SUB + (ROWS_SUB // CHUNK) * CHUNK, tail)])

    plsc.subcore_barrier()

    pltpu.sync_copy(srcr.at[pl.ds(s * EDGES_SUB, EDGES_SUB)], srcf)

    def gat_start(chunk, buf, sem_):
        idx = srcf.at[pl.ds(chunk * CHUNK, CHUNK)]

        @pl.when(c == 0)
        def _():
            pltpu.async_copy(ha.at[idx], buf, sem_)

        @pl.when(c == 1)
        def _():
            pltpu.async_copy(hb.at[idx], buf, sem_)

    def gat_wait(chunk, buf, sem_):
        pltpu.make_async_copy(
            ha.at[srcf.at[pl.ds(chunk * CHUNK, CHUNK)]], buf, sem_).wait()

    def sca_start(t, buf, sem_):
        pltpu.async_copy(buf, aggsh.at[dpage.at[t]], sem_, add=True)

    def sca_wait(buf, sem_):
        pltpu.make_async_copy(buf, aggsh.at[dpage.at[0]], sem_).wait()

    gat_start(0, rows[0], sg[0])
    gat_start(1, rows[1], sg[1])
    gat_start(2, rows[2], sg[2])

    @pl.loop(0, NCHUNK_SUB, step=4)
    def _(k):
        t0 = jnp.bitwise_and(k, 7)

        @pl.when(t0 == 0)
        def _():
            km = pl.multiple_of(k, 8)
            pltpu.sync_copy(dstr.at[pl.ds(s * NCHUNK_SUB + km, 8)], dpage)

        for t in range(4):
            j = k + t
            gat_wait(j, rows[t], sg[t])
            sca_start(t0 + t, rows[t], ss[t])
            tp = (t + 3) % 4

            @pl.when(j >= 1)
            def _():
                sca_wait(rows[tp], ss[tp])

            @pl.when(j < NCHUNK_SUB - 3)
            def _():
                gat_start(j + 3, rows[tp], sg[tp])

    sca_wait(rows[3], ss[3])

    plsc.subcore_barrier()

    @pl.when(jnp.logical_and(c == 0, s < 15))
    def _():
        pltpu.sync_copy(aggsh.at[pl.ds(s * ROWS_SUB, ROWS_SUB)],
                        agga.at[pl.ds(s * ROWS_SUB, ROWS_SUB)])

    @pl.when(jnp.logical_and(c == 0, s == 15))
    def _():
        pltpu.sync_copy(aggsh.at[pl.ds(15 * ROWS_SUB, LAST_ROWS)],
                        agga.at[pl.ds(15 * ROWS_SUB, LAST_ROWS)])

    @pl.when(jnp.logical_and(c == 1, s < 15))
    def _():
        pltpu.sync_copy(aggsh.at[pl.ds(s * ROWS_SUB, ROWS_SUB)],
                        aggb.at[pl.ds(s * ROWS_SUB, ROWS_SUB)])

    @pl.when(jnp.logical_and(c == 1, s == 15))
    def _():
        pltpu.sync_copy(aggsh.at[pl.ds(15 * ROWS_SUB, LAST_ROWS)],
                        aggb.at[pl.ds(15 * ROWS_SUB, LAST_ROWS)])


def _segsum(ha, hb, src2d, dst2d, zeros_init):
    mesh = plsc.VectorSubcoreMesh(core_axis_name="c", subcore_axis_name="s")
    f = pl.kernel(
        _segsum_body,
        mesh=mesh,
        out_type=[
            jax.ShapeDtypeStruct((N, HALF), jnp.float32),
            jax.ShapeDtypeStruct((N, HALF), jnp.float32),
        ],
        scratch_types=[
            pltpu.VMEM((EDGES_SUB,), jnp.int32),
            pltpu.VMEM((8, CHUNK), jnp.int32),
        ]
        + [pltpu.VMEM((CHUNK, HALF), jnp.float32)] * 4
        + [pltpu.VMEM_SHARED((AGG_ROWS, HALF), jnp.float32)]
        + [pltpu.SemaphoreType.DMA] * 8,
    )
    return f(ha, hb, src2d, dst2d, zeros_init)


def _mlpbn_common(p, j, ha_ref, hb_ref, aa_ref, ab_ref, w1_ref, b1_ref,
                  w2_ref, b2_ref, g_ref, be_ref, y_scr, sm_scr, sq_scr):
    @pl.when(p == 0)
    def _():
        z = jnp.concatenate(
            [ha_ref[...] + aa_ref[...], hb_ref[...] + ab_ref[...]], axis=1)
        t = jnp.dot(z.astype(jnp.bfloat16), w1_ref[...],
                    preferred_element_type=jnp.float32)
        t = jnp.maximum(t + b1_ref[...], 0.0)
        y = jnp.dot(t.astype(jnp.bfloat16), w2_ref[...],
                    preferred_element_type=jnp.float32)
        y = y + b2_ref[...]
        y_scr[j] = y
        ps = jnp.sum(y, axis=0, keepdims=True)
        pq = jnp.sum(y * y, axis=0, keepdims=True)

        @pl.when(j == 0)
        def _():
            sm_scr[...] = ps
            sq_scr[...] = pq

        @pl.when(j > 0)
        def _():
            sm_scr[...] += ps
            sq_scr[...] += pq

    mean = sm_scr[...] * (1.0 / N)
    var = sq_scr[...] * (1.0 / N) - mean * mean
    inv = g_ref[...] / jnp.sqrt(var + BN_EPS)
    return jnp.maximum((y_scr[j] - mean) * inv + be_ref[...], 0.0)


def _mlpbn_mid_body(ha_ref, hb_ref, aa_ref, ab_ref, w1_ref, b1_ref, w2_ref,
                    b2_ref, g_ref, be_ref, hao_ref, hbo_ref,
                    y_scr, sm_scr, sq_scr):
    p = pl.program_id(0)
    j = pl.program_id(1)
    h = _mlpbn_common(p, j, ha_ref, hb_ref, aa_ref, ab_ref, w1_ref, b1_ref,
                      w2_ref, b2_ref, g_ref, be_ref, y_scr, sm_scr, sq_scr)

    @pl.when(p == 1)
    def _():
        hao_ref[...] = h[:, :HALF]
        hbo_ref[...] = h[:, HALF:]


def _mlpbn_final_body(ha_ref, hb_ref, aa_ref, ab_ref, w1_ref, b1_ref, w2_ref,
                      b2_ref, g_ref, be_ref, h0a_ref, h0b_ref, out_ref,
                      y_scr, sm_scr, sq_scr):
    p = pl.program_id(0)
    j = pl.program_id(1)
    h = _mlpbn_common(p, j, ha_ref, hb_ref, aa_ref, ab_ref, w1_ref, b1_ref,
                      w2_ref, b2_ref, g_ref, be_ref, y_scr, sm_scr, sq_scr)

    @pl.when(p == 1)
    def _():
        res = jnp.concatenate([h0a_ref[...], h0b_ref[...]], axis=1)
        out_ref[...] = h + res


def _p0_blk(p, j):
    return ((1 - p) * j, 0)


def _p1_blk(p, j):
    return (p * j, 0)


def _whole(p, j):
    return (0, 0)


_COMMON_SPECS = [
    pl.BlockSpec((BLK, HALF), _p0_blk),
    pl.BlockSpec((BLK, HALF), _p0_blk),
    pl.BlockSpec((BLK, HALF), _p0_blk),
    pl.BlockSpec((BLK, HALF), _p0_blk),
    pl.BlockSpec((HID, HID), _whole),
    pl.BlockSpec((1, HID), _whole),
    pl.BlockSpec((HID, HID), _whole),
    pl.BlockSpec((1, HID), _whole),
    pl.BlockSpec((1, HID), _whole),
    pl.BlockSpec((1, HID), _whole),
]

_SCRATCH = [
    pltpu.VMEM((NBLK, BLK, HID), jnp.float32),
    pltpu.VMEM((1, HID), jnp.float32),
    pltpu.VMEM((1, HID), jnp.float32),
]


def _mlpbn_mid(ha, hb, aa, ab, W1, b12, W2, b22, g2, be2):
    return pl.pallas_call(
        _mlpbn_mid_body,
        grid=(2, NBLK),
        in_specs=_COMMON_SPECS,
        out_specs=[
            pl.BlockSpec((BLK, HALF), lambda p, j: (j, 0)),
            pl.BlockSpec((BLK, HALF), lambda p, j: (j, 0)),
        ],
        out_shape=[
            jax.ShapeDtypeStruct((N, HALF), jnp.float32),
            jax.ShapeDtypeStruct((N, HALF), jnp.float32),
        ],
        scratch_shapes=_SCRATCH,
    )(ha, hb, aa, ab, W1, b12, W2, b22, g2, be2)


def _mlpbn_final(ha, hb, aa, ab, W1, b12, W2, b22, g2, be2, h0a, h0b):
    return pl.pallas_call(
        _mlpbn_final_body,
        grid=(2, NBLK),
        in_specs=_COMMON_SPECS + [
            pl.BlockSpec((BLK, HALF), _p1_blk),
            pl.BlockSpec((BLK, HALF), _p1_blk),
        ],
        out_specs=pl.BlockSpec((BLK, HID), lambda p, j: (j, 0)),
        out_shape=jax.ShapeDtypeStruct((N, HID), jnp.float32),
        scratch_shapes=_SCRATCH,
    )(ha, hb, aa, ab, W1, b12, W2, b22, g2, be2, h0a, h0b)


def kernel(x, edge_index, residual, Wp, bp, W1, b1, W2, b2, gamma, beta):
    npad = E_PAD - E
    pad_src = jnp.zeros((npad,), jnp.int32)
    pad_dst = N + (jnp.arange(npad, dtype=jnp.int32) % (AGG_ROWS - N))
    src2d = jnp.concatenate([edge_index[0], pad_src])
    dst2d = jnp.concatenate([edge_index[1], pad_dst]).reshape(
        NCHUNK_TOTAL, CHUNK)
    zeros_init = jnp.zeros((CHUNK, HALF), jnp.float32)
    Wp = Wp.astype(jnp.bfloat16)
    W1 = W1.astype(jnp.bfloat16)
    W2 = W2.astype(jnp.bfloat16)
    bp2 = bp.reshape(1, HID)
    b12 = b1.reshape(1, HID)
    b22 = b2.reshape(1, HID)
    g2 = gamma.reshape(1, HID)
    be2 = beta.reshape(1, HID)

    ha0, hb0 = _proj(x, Wp, bp2)
    ha, hb = ha0, hb0

    aa, ab = _segsum(ha, hb, src2d, dst2d, zeros_init)
    ha, hb = _mlpbn_mid(ha, hb, aa, ab, W1, b12, W2, b22, g2, be2)

    aa, ab = _segsum(ha, hb, src2d, dst2d, zeros_init)
    return _mlpbn_final(ha, hb, aa, ab, W1, b12, W2, b22, g2, be2, ha0, hb0)


# speedup vs baseline: 1.0440x; 1.0440x over previous
import functools

import jax
import jax.numpy as jnp
from jax import lax
from jax.experimental import pallas as pl
from jax.experimental.pallas import tpu as pltpu
from jax.experimental.pallas import tpu_sc as plsc

N = 10000
E = 160000
HID = 256
HALF = 128
BN_EPS = 1e-5

BLK = 1000
NBLK = N // BLK

CHUNK = 64
NSUB = 16
NCHUNK_SUB = 160
NCHUNK_TOTAL = NSUB * NCHUNK_SUB
E_PAD = NCHUNK_TOTAL * CHUNK
EDGES_SUB = NCHUNK_SUB * CHUNK
ROWS_SUB = 632
AGG_ROWS = NSUB * ROWS_SUB
LAST_ROWS = N - 15 * ROWS_SUB


def _proj_body(x_ref, wp_ref, bp_ref, ha_ref, hb_ref):
    h = jnp.dot(x_ref[...].astype(jnp.bfloat16), wp_ref[...],
                preferred_element_type=jnp.float32)
    h = jnp.maximum(h + bp_ref[...], 0.0)
    ha_ref[...] = h[:, :HALF]
    hb_ref[...] = h[:, HALF:]


def _proj(x, Wp, bp2):
    return pl.pallas_call(
        _proj_body,
        grid=(NBLK,),
        in_specs=[
            pl.BlockSpec((BLK, HID), lambda i: (i, 0)),
            pl.BlockSpec((HID, HID), lambda i: (0, 0)),
            pl.BlockSpec((1, HID), lambda i: (0, 0)),
        ],
        out_specs=[
            pl.BlockSpec((BLK, HALF), lambda i: (i, 0)),
            pl.BlockSpec((BLK, HALF), lambda i: (i, 0)),
        ],
        out_shape=[
            jax.ShapeDtypeStruct((N, HALF), jnp.float32),
            jax.ShapeDtypeStruct((N, HALF), jnp.float32),
        ],
    )(x, Wp, bp2)


def _segsum_body(ha, hb, srcr, dstr, zr, agga, aggb,
                 srcf, dpage, *rest):
    rows = rest[0:4]
    aggsh = rest[4]
    sg = rest[5:9]
    ss = rest[9:13]
    c = lax.axis_index("c")
    s = lax.axis_index("s")

    pltpu.sync_copy(zr, aggsh.at[pl.ds(s * ROWS_SUB, ROWS_SUB)])

    plsc.subcore_barrier()

    pltpu.sync_copy(srcr.at[pl.ds(s * EDGES_SUB, EDGES_SUB)], srcf)

    def gat_start(chunk, buf, sem_):
        idx = srcf.at[pl.ds(chunk * CHUNK, CHUNK)]

        @pl.when(c == 0)
        def _():
            pltpu.async_copy(ha.at[idx], buf, sem_)

        @pl.when(c == 1)
        def _():
            pltpu.async_copy(hb.at[idx], buf, sem_)

    def gat_wait(chunk, buf, sem_):
        pltpu.make_async_copy(
            ha.at[srcf.at[pl.ds(chunk * CHUNK, CHUNK)]], buf, sem_).wait()

    def sca_start(t, buf, sem_):
        pltpu.async_copy(buf, aggsh.at[dpage.at[t]], sem_, add=True)

    def sca_wait(buf, sem_):
        pltpu.make_async_copy(buf, aggsh.at[dpage.at[0]], sem_).wait()

    gat_start(0, rows[0], sg[0])
    gat_start(1, rows[1], sg[1])
    gat_start(2, rows[2], sg[2])

    @pl.loop(0, NCHUNK_SUB, step=4)
    def _(k):
        t0 = jnp.bitwise_and(k, 7)

        @pl.when(t0 == 0)
        def _():
            km = pl.multiple_of(k, 8)
            pltpu.sync_copy(dstr.at[pl.ds(s * NCHUNK_SUB + km, 8)], dpage)

        for t in range(4):
            j = k + t
            gat_wait(j, rows[t], sg[t])
            sca_start(t0 + t, rows[t], ss[t])
            tp = (t + 3) % 4

            @pl.when(j >= 1)
            def _():
                sca_wait(rows[tp], ss[tp])

            @pl.when(j < NCHUNK_SUB - 3)
            def _():
                gat_start(j + 3, rows[tp], sg[tp])

    sca_wait(rows[3], ss[3])

    plsc.subcore_barrier()

    @pl.when(jnp.logical_and(c == 0, s < 15))
    def _():
        pltpu.sync_copy(aggsh.at[pl.ds(s * ROWS_SUB, ROWS_SUB)],
                        agga.at[pl.ds(s * ROWS_SUB, ROWS_SUB)])

    @pl.when(jnp.logical_and(c == 0, s == 15))
    def _():
        pltpu.sync_copy(aggsh.at[pl.ds(15 * ROWS_SUB, LAST_ROWS)],
                        agga.at[pl.ds(15 * ROWS_SUB, LAST_ROWS)])

    @pl.when(jnp.logical_and(c == 1, s < 15))
    def _():
        pltpu.sync_copy(aggsh.at[pl.ds(s * ROWS_SUB, ROWS_SUB)],
                        aggb.at[pl.ds(s * ROWS_SUB, ROWS_SUB)])

    @pl.when(jnp.logical_and(c == 1, s == 15))
    def _():
        pltpu.sync_copy(aggsh.at[pl.ds(15 * ROWS_SUB, LAST_ROWS)],
                        aggb.at[pl.ds(15 * ROWS_SUB, LAST_ROWS)])


def _segsum(ha, hb, src2d, dst2d, zeros_init):
    mesh = plsc.VectorSubcoreMesh(core_axis_name="c", subcore_axis_name="s")
    f = pl.kernel(
        _segsum_body,
        mesh=mesh,
        out_type=[
            jax.ShapeDtypeStruct((N, HALF), jnp.float32),
            jax.ShapeDtypeStruct((N, HALF), jnp.float32),
        ],
        scratch_types=[
            pltpu.VMEM((EDGES_SUB,), jnp.int32),
            pltpu.VMEM((8, CHUNK), jnp.int32),
        ]
        + [pltpu.VMEM((CHUNK, HALF), jnp.float32)] * 4
        + [pltpu.VMEM_SHARED((AGG_ROWS, HALF), jnp.float32)]
        + [pltpu.SemaphoreType.DMA] * 8,
    )
    return f(ha, hb, src2d, dst2d, zeros_init)


def _mlpbn_common(p, j, ha_ref, hb_ref, aa_ref, ab_ref, w1_ref, b1_ref,
                  w2_ref, b2_ref, g_ref, be_ref, y_scr, sm_scr, sq_scr):
    @pl.when(p == 0)
    def _():
        z = jnp.concatenate(
            [ha_ref[...] + aa_ref[...], hb_ref[...] + ab_ref[...]], axis=1)
        t = jnp.dot(z.astype(jnp.bfloat16), w1_ref[...],
                    preferred_element_type=jnp.float32)
        t = jnp.maximum(t + b1_ref[...], 0.0)
        y = jnp.dot(t.astype(jnp.bfloat16), w2_ref[...],
                    preferred_element_type=jnp.float32)
        y = y + b2_ref[...]
        y_scr[j] = y
        ps = jnp.sum(y, axis=0, keepdims=True)
        pq = jnp.sum(y * y, axis=0, keepdims=True)

        @pl.when(j == 0)
        def _():
            sm_scr[...] = ps
            sq_scr[...] = pq

        @pl.when(j > 0)
        def _():
            sm_scr[...] += ps
            sq_scr[...] += pq

    mean = sm_scr[...] * (1.0 / N)
    var = sq_scr[...] * (1.0 / N) - mean * mean
    inv = g_ref[...] / jnp.sqrt(var + BN_EPS)
    return jnp.maximum((y_scr[j] - mean) * inv + be_ref[...], 0.0)


def _mlpbn_mid_body(ha_ref, hb_ref, aa_ref, ab_ref, w1_ref, b1_ref, w2_ref,
                    b2_ref, g_ref, be_ref, hao_ref, hbo_ref,
                    y_scr, sm_scr, sq_scr):
    p = pl.program_id(0)
    j = pl.program_id(1)
    h = _mlpbn_common(p, j, ha_ref, hb_ref, aa_ref, ab_ref, w1_ref, b1_ref,
                      w2_ref, b2_ref, g_ref, be_ref, y_scr, sm_scr, sq_scr)

    @pl.when(p == 1)
    def _():
        hao_ref[...] = h[:, :HALF]
        hbo_ref[...] = h[:, HALF:]


def _mlpbn_final_body(ha_ref, hb_ref, aa_ref, ab_ref, w1_ref, b1_ref, w2_ref,
                      b2_ref, g_ref, be_ref, h0a_ref, h0b_ref, out_ref,
                      y_scr, sm_scr, sq_scr):
    p = pl.program_id(0)
    j = pl.program_id(1)
    h = _mlpbn_common(p, j, ha_ref, hb_ref, aa_ref, ab_ref, w1_ref, b1_ref,
                      w2_ref, b2_ref, g_ref, be_ref, y_scr, sm_scr, sq_scr)

    @pl.when(p == 1)
    def _():
        res = jnp.concatenate([h0a_ref[...], h0b_ref[...]], axis=1)
        out_ref[...] = h + res


def _p0_blk(p, j):
    return ((1 - p) * j, 0)


def _p1_blk(p, j):
    return (p * j, 0)


def _whole(p, j):
    return (0, 0)


_COMMON_SPECS = [
    pl.BlockSpec((BLK, HALF), _p0_blk),
    pl.BlockSpec((BLK, HALF), _p0_blk),
    pl.BlockSpec((BLK, HALF), _p0_blk),
    pl.BlockSpec((BLK, HALF), _p0_blk),
    pl.BlockSpec((HID, HID), _whole),
    pl.BlockSpec((1, HID), _whole),
    pl.BlockSpec((HID, HID), _whole),
    pl.BlockSpec((1, HID), _whole),
    pl.BlockSpec((1, HID), _whole),
    pl.BlockSpec((1, HID), _whole),
]

_SCRATCH = [
    pltpu.VMEM((NBLK, BLK, HID), jnp.float32),
    pltpu.VMEM((1, HID), jnp.float32),
    pltpu.VMEM((1, HID), jnp.float32),
]


def _mlpbn_mid(ha, hb, aa, ab, W1, b12, W2, b22, g2, be2):
    return pl.pallas_call(
        _mlpbn_mid_body,
        grid=(2, NBLK),
        in_specs=_COMMON_SPECS,
        out_specs=[
            pl.BlockSpec((BLK, HALF), lambda p, j: (j, 0)),
            pl.BlockSpec((BLK, HALF), lambda p, j: (j, 0)),
        ],
        out_shape=[
            jax.ShapeDtypeStruct((N, HALF), jnp.float32),
            jax.ShapeDtypeStruct((N, HALF), jnp.float32),
        ],
        scratch_shapes=_SCRATCH,
    )(ha, hb, aa, ab, W1, b12, W2, b22, g2, be2)


def _mlpbn_final(ha, hb, aa, ab, W1, b12, W2, b22, g2, be2, h0a, h0b):
    return pl.pallas_call(
        _mlpbn_final_body,
        grid=(2, NBLK),
        in_specs=_COMMON_SPECS + [
            pl.BlockSpec((BLK, HALF), _p1_blk),
            pl.BlockSpec((BLK, HALF), _p1_blk),
        ],
        out_specs=pl.BlockSpec((BLK, HID), lambda p, j: (j, 0)),
        out_shape=jax.ShapeDtypeStruct((N, HID), jnp.float32),
        scratch_shapes=_SCRATCH,
    )(ha, hb, aa, ab, W1, b12, W2, b22, g2, be2, h0a, h0b)


def kernel(x, edge_index, residual, Wp, bp, W1, b1, W2, b2, gamma, beta):
    npad = E_PAD - E
    pad_src = jnp.zeros((npad,), jnp.int32)
    pad_dst = N + (jnp.arange(npad, dtype=jnp.int32) % (AGG_ROWS - N))
    src2d = jnp.concatenate([edge_index[0], pad_src])
    dst2d = jnp.concatenate([edge_index[1], pad_dst]).reshape(
        NCHUNK_TOTAL, CHUNK)
    zeros_init = jnp.zeros((ROWS_SUB, HALF), jnp.float32)
    Wp = Wp.astype(jnp.bfloat16)
    W1 = W1.astype(jnp.bfloat16)
    W2 = W2.astype(jnp.bfloat16)
    bp2 = bp.reshape(1, HID)
    b12 = b1.reshape(1, HID)
    b22 = b2.reshape(1, HID)
    g2 = gamma.reshape(1, HID)
    be2 = beta.reshape(1, HID)

    ha0, hb0 = _proj(x, Wp, bp2)
    ha, hb = ha0, hb0

    aa, ab = _segsum(ha, hb, src2d, dst2d, zeros_init)
    ha, hb = _mlpbn_mid(ha, hb, aa, ab, W1, b12, W2, b22, g2, be2)

    aa, ab = _segsum(ha, hb, src2d, dst2d, zeros_init)
    return _mlpbn_final(ha, hb, aa, ab, W1, b12, W2, b22, g2, be2, ha0, hb0)
